# baseline (speedup 1.0000x reference)
import jax
import jax.numpy as jnp
from jax import lax
from jax.experimental import pallas as pl
from jax.experimental.pallas import tpu as pltpu
from jax.experimental.pallas import tpu_sc as plsc

NC = 2
NS = 16
NW = NC * NS
L = 16
BB = 128
XP = 129
CW = 64
CP = 65


def _fmt_body(tabt_hbm, tabr_hbm, slab, xp, dsems, wsems):
    D = tabt_hbm.shape[0]
    V = tabt_hbm.shape[1]
    nch = V // CW
    wid = lax.axis_index("s") * NC + lax.axis_index("c")
    kmax = (nch + NW - 1) // NW
    kmax = kmax + (kmax % 2)

    col_ids = [lax.iota(jnp.int32, L) + c * L for c in range(CW // L)]

    def chunk(k):
        return wid + k * NW

    def in_slice(k):
        return tabt_hbm.at[:, pl.ds(chunk(k) * CW, CW)]

    def dma_start(k, b):
        pltpu.async_copy(in_slice(k), slab[b], dsems[b])

    def dma_wait(k, b):
        pltpu.make_async_copy(in_slice(k), slab[b], dsems[b]).wait()

    def out_slice(k):
        return tabr_hbm.at[pl.ds(chunk(k) * CW, CW)]

    def write_start(k, b):
        pltpu.async_copy(xp[b].at[:, pl.ds(0, D)], out_slice(k), wsems[b])

    def write_drain(k, b):
        pltpu.make_async_copy(xp[b].at[:, pl.ds(0, D)], out_slice(k),
                              wsems[b]).wait()

    @pl.when(chunk(0) < nch)
    def _():
        dma_start(0, 0)

    @pl.loop(0, kmax, step=2)
    def block(k0):
        for j in range(2):
            b = j % 2
            k = k0 + j

            @pl.when(chunk(k + 1) < nch)
            def _():
                dma_start(k + 1, (j + 1) % 2)

            @pl.when(jnp.logical_and(k >= 2, chunk(k - 2) < nch))
            def _():
                write_drain(k - 2, b)

            @pl.when(chunk(k) < nch)
            def _():
                dma_wait(k, b)

                @plsc.parallel_loop(0, D, step=1, unroll=4)
                def transpose_row(d):
                    dv = jnp.full((L,), d, dtype=jnp.int32)
                    for c in range(CW // L):
                        v = slab[b][d, pl.ds(c * L, L)]
                        plsc.store_scatter(xp[b], [col_ids[c], dv], v)

                write_start(k, b)

    for j in range(2):
        k = kmax - 2 + j

        @pl.when(chunk(k) < nch)
        def _():
            write_drain(k, j)


def _emb_body(tokt_hbm, table_hbm, out_hbm, idx_v, rows, xps,
              tsems, gsems, wsems):
    D = table_hbm.shape[1]
    scale = float(D) ** 0.5
    ncg = D // 8
    nbb = out_hbm.shape[2]
    n_items = tokt_hbm.shape[0] // BB
    ipw = n_items // NW
    wid = lax.axis_index("s") * NC + lax.axis_index("c")
    m0 = wid * ipw

    col_ids = [lax.iota(jnp.int32, L) + c * L for c in range(D // L)]

    def tok_slice(m):
        return tokt_hbm.at[pl.ds(m * BB, BB)]

    def tok_start(m, s):
        pltpu.async_copy(tok_slice(m), idx_v.at[s], tsems[s])

    def tok_wait(s):
        pltpu.make_async_copy(tok_slice(0), idx_v.at[s], tsems[s]).wait()

    def gather_start(b, s):
        pltpu.async_copy(table_hbm.at[idx_v.at[s]], rows[b], gsems[b])

    def gather_wait(b, s):
        pltpu.make_async_copy(table_hbm.at[idx_v.at[s]], rows[b],
                              gsems[b]).wait()

    def write_start(m, x):
        t = m // nbb
        bb = m % nbb
        for cg in range(ncg):
            pltpu.async_copy(xps[x].at[pl.ds(cg * 8, 8), pl.ds(0, BB)],
                             out_hbm.at[t, cg, bb], wsems[x])

    def write_drain(x):
        for cg in range(ncg):
            pltpu.make_async_copy(xps[x].at[pl.ds(cg * 8, 8), pl.ds(0, BB)],
                                  out_hbm.at[0, cg, 0], wsems[x]).wait()

    for j in range(3):
        tok_start(m0 + j, j)
    tok_wait(0)
    gather_start(0, 0)

    @pl.loop(0, ipw, step=4)
    def block(k0):
        for j in range(4):
            m = k0 + j
            b = j % 2
            x = j % 2
            s = j % 4

            @pl.when(m + 3 < ipw)
            def _():
                tok_start(m0 + m + 3, (j + 3) % 4)

            @pl.when(m + 1 < ipw)
            def _():
                tok_wait((j + 1) % 4)
                gather_start((j + 1) % 2, (j + 1) % 4)

            gather_wait(b, s)

            @pl.when(m >= 2)
            def _():
                write_drain(x)

            @plsc.parallel_loop(0, BB, step=1, unroll=4)
            def transpose_row(r):
                rv = jnp.full((L,), r, dtype=jnp.int32)
                for c in range(D // L):
                    v = rows[b][r, pl.ds(c * L, L)]
                    plsc.store_scatter(xps[x], [col_ids[c], rv], v * scale)

            write_start(m0 + m, x)

    for x in range(2):
        write_drain(x)


def kernel(tokens, table):
    Bdim, T = tokens.shape
    V, D = table.shape
    nbb = Bdim // BB
    tokt = jnp.transpose(tokens).reshape(-1).astype(jnp.int32)
    tabt = jnp.transpose(table)
    mesh = plsc.VectorSubcoreMesh(
        core_axis_name="c", subcore_axis_name="s",
        num_cores=NC, num_subcores=NS,
    )
    tabr = pl.kernel(
        _fmt_body,
        out_type=jax.ShapeDtypeStruct((V, D), table.dtype),
        mesh=mesh,
        scratch_types=[
            [pltpu.VMEM((D, CW), jnp.float32) for _ in range(2)],
            [pltpu.VMEM((CW, CP), jnp.float32) for _ in range(2)],
            [pltpu.SemaphoreType.DMA for _ in range(2)],
            [pltpu.SemaphoreType.DMA for _ in range(2)],
        ],
        compiler_params=pltpu.CompilerParams(
            use_tc_tiling_on_sc=False, needs_layout_passes=False),
    )(tabt)
    out5 = pl.kernel(
        _emb_body,
        out_type=jax.ShapeDtypeStruct((T, D // 8, nbb, 8, BB), table.dtype),
        mesh=mesh,
        scratch_types=[
            pltpu.VMEM((4, BB), jnp.int32),
            [pltpu.VMEM((BB, D), jnp.float32) for _ in range(2)],
            [pltpu.VMEM((D, XP), jnp.float32) for _ in range(2)],
            [pltpu.SemaphoreType.DMA for _ in range(4)],
            [pltpu.SemaphoreType.DMA for _ in range(2)],
            [pltpu.SemaphoreType.DMA for _ in range(2)],
        ],
        compiler_params=pltpu.CompilerParams(
            use_tc_tiling_on_sc=False, needs_layout_passes=False),
    )(tokt, tabr)
    return out5.transpose(2, 4, 0, 1, 3).reshape(Bdim, T, D)


# speedup vs baseline: 7.0300x; 7.0300x over previous
import jax
import jax.numpy as jnp
from jax import lax
from jax.experimental import pallas as pl
from jax.experimental.pallas import tpu as pltpu
from jax.experimental.pallas import tpu_sc as plsc

NC = 2
NS = 16
NW = NC * NS
L = 16
BB = 128
XP = 129


def _emb_body(tokt_hbm, table_hbm, out_hbm, idx_v, rows, xps,
              tsems, gsems, wsems):
    D = table_hbm.shape[1]
    scale = float(D) ** 0.5
    ncg = D // 8
    nbb = out_hbm.shape[2]
    n_items = tokt_hbm.shape[0] // BB
    ipw = n_items // NW
    wid = lax.axis_index("s") * NC + lax.axis_index("c")
    m0 = wid * ipw

    col_ids = [lax.iota(jnp.int32, L) + c * L for c in range(D // L)]

    def tok_slice(m):
        return tokt_hbm.at[pl.ds(m * BB, BB)]

    def tok_start(m, s):
        pltpu.async_copy(tok_slice(m), idx_v.at[s], tsems[s])

    def tok_wait(s):
        pltpu.make_async_copy(tok_slice(0), idx_v.at[s], tsems[s]).wait()

    def gather_start(b, s):
        pltpu.async_copy(table_hbm.at[idx_v.at[s]], rows[b], gsems[b])

    def gather_wait(b, s):
        pltpu.make_async_copy(table_hbm.at[idx_v.at[s]], rows[b],
                              gsems[b]).wait()

    def write_start(m, x):
        t = m // nbb
        bb = m % nbb
        for cg in range(ncg):
            pltpu.async_copy(xps[x].at[pl.ds(cg * 8, 8), pl.ds(0, BB)],
                             out_hbm.at[t, cg, bb], wsems[x])

    def write_drain(x):
        for cg in range(ncg):
            pltpu.make_async_copy(xps[x].at[pl.ds(cg * 8, 8), pl.ds(0, BB)],
                                  out_hbm.at[0, cg, 0], wsems[x]).wait()

    for j in range(3):
        tok_start(m0 + j, j)
    tok_wait(0)
    gather_start(0, 0)

    @pl.loop(0, ipw, step=4)
    def block(k0):
        for j in range(4):
            m = k0 + j
            b = j % 2
            x = j % 2
            s = j % 4

            @pl.when(m + 3 < ipw)
            def _():
                tok_start(m0 + m + 3, (j + 3) % 4)

            @pl.when(m + 1 < ipw)
            def _():
                tok_wait((j + 1) % 4)
                gather_start((j + 1) % 2, (j + 1) % 4)

            gather_wait(b, s)

            @pl.when(m >= 2)
            def _():
                write_drain(x)

            @plsc.parallel_loop(0, BB, step=1, unroll=4)
            def transpose_row(r):
                rv = jnp.full((L,), r, dtype=jnp.int32)
                for c in range(D // L):
                    v = rows[b][r, pl.ds(c * L, L)]
                    plsc.store_scatter(xps[x], [col_ids[c], rv], v * scale)

            write_start(m0 + m, x)

    for x in range(2):
        write_drain(x)


def kernel(tokens, table):
    Bdim, T = tokens.shape
    V, D = table.shape
    nbb = Bdim // BB
    tokt = jnp.transpose(tokens).reshape(-1).astype(jnp.int32)
    mesh = plsc.VectorSubcoreMesh(
        core_axis_name="c", subcore_axis_name="s",
        num_cores=NC, num_subcores=NS,
    )
    out5 = pl.kernel(
        _emb_body,
        out_type=jax.ShapeDtypeStruct((T, D // 8, nbb, 8, BB), table.dtype),
        mesh=mesh,
        scratch_types=[
            pltpu.VMEM((4, BB), jnp.int32),
            [pltpu.VMEM((BB, D), jnp.float32) for _ in range(2)],
            [pltpu.VMEM((D, XP), jnp.float32) for _ in range(2)],
            [pltpu.SemaphoreType.DMA for _ in range(4)],
            [pltpu.SemaphoreType.DMA for _ in range(2)],
            [pltpu.SemaphoreType.DMA for _ in range(2)],
        ],
        compiler_params=pltpu.CompilerParams(
            use_tc_tiling_on_sc=False, needs_layout_passes=False),
    )(tokt, table)
    return out5.transpose(2, 4, 0, 1, 3).reshape(Bdim, T, D)


# speedup vs baseline: 7.3256x; 1.0420x over previous
import jax
import jax.numpy as jnp
from jax import lax
from jax.experimental import pallas as pl
from jax.experimental.pallas import tpu as pltpu
from jax.experimental.pallas import tpu_sc as plsc

NC = 2
NS = 16
NW = NC * NS
L = 16
BB = 128
XP = 129
NB = 4


def _emb_body(tokt_hbm, table_hbm, out_hbm, idx_v, rows, xps,
              tsems, gsems, wsems):
    D = table_hbm.shape[1]
    scale = float(D) ** 0.5
    ncg = D // 8
    nbb = out_hbm.shape[2]
    n_items = tokt_hbm.shape[0] // BB
    ipw = n_items // NW
    wid = lax.axis_index("s") * NC + lax.axis_index("c")
    m0 = wid * ipw

    col_ids = [lax.iota(jnp.int32, L) + c * L for c in range(D // L)]

    def tok_slice(m):
        return tokt_hbm.at[pl.ds(m * BB, BB)]

    def tok_start(m, s):
        pltpu.async_copy(tok_slice(m), idx_v.at[s], tsems[s])

    def tok_wait(s):
        pltpu.make_async_copy(tok_slice(0), idx_v.at[s], tsems[s]).wait()

    def gather_start(s):
        pltpu.async_copy(table_hbm.at[idx_v.at[s]], rows[s], gsems[s])

    def gather_wait(s):
        pltpu.make_async_copy(table_hbm.at[idx_v.at[s]], rows[s],
                              gsems[s]).wait()

    def write_start(m, x):
        t = m // nbb
        bb = m % nbb
        for cg in range(ncg):
            pltpu.async_copy(xps[x].at[pl.ds(cg * 8, 8), pl.ds(0, BB)],
                             out_hbm.at[t, cg, bb], wsems[x])

    def write_drain(x):
        for cg in range(ncg):
            pltpu.make_async_copy(xps[x].at[pl.ds(cg * 8, 8), pl.ds(0, BB)],
                                  out_hbm.at[0, cg, 0], wsems[x]).wait()

    for j in range(NB):
        tok_start(m0 + j, j)
    for j in range(NB - 1):
        tok_wait(j)
        gather_start(j)

    @pl.loop(0, ipw, step=NB)
    def block(k0):
        for j in range(NB):
            m = k0 + j

            gather_wait(j)

            @pl.when(m + NB < ipw)
            def _():
                tok_start(m0 + m + NB, j)

            @pl.when(m >= NB)
            def _():
                write_drain(j)

            @plsc.parallel_loop(0, BB, step=1, unroll=4)
            def transpose_row(r):
                rv = jnp.full((L,), r, dtype=jnp.int32)
                for c in range(D // L):
                    v = rows[j][r, pl.ds(c * L, L)]
                    plsc.store_scatter(xps[j], [col_ids[c], rv], v * scale)

            write_start(m0 + m, j)

            @pl.when(m + NB - 1 < ipw)
            def _():
                tok_wait((j + NB - 1) % NB)
                gather_start((j + NB - 1) % NB)

    for x in range(NB):
        write_drain(x)


def kernel(tokens, table):
    Bdim, T = tokens.shape
    V, D = table.shape
    nbb = Bdim // BB
    tokt = jnp.transpose(tokens).reshape(-1).astype(jnp.int32)
    mesh = plsc.VectorSubcoreMesh(
        core_axis_name="c", subcore_axis_name="s",
        num_cores=NC, num_subcores=NS,
    )
    out5 = pl.kernel(
        _emb_body,
        out_type=jax.ShapeDtypeStruct((T, D // 8, nbb, 8, BB), table.dtype),
        mesh=mesh,
        scratch_types=[
            pltpu.VMEM((NB, BB), jnp.int32),
            [pltpu.VMEM((BB, D), jnp.float32) for _ in range(NB)],
            [pltpu.VMEM((D, XP), jnp.float32) for _ in range(NB)],
            [pltpu.SemaphoreType.DMA for _ in range(NB)],
            [pltpu.SemaphoreType.DMA for _ in range(NB)],
            [pltpu.SemaphoreType.DMA for _ in range(NB)],
        ],
        compiler_params=pltpu.CompilerParams(
            use_tc_tiling_on_sc=False, needs_layout_passes=False),
    )(tokt, table)
    return out5.transpose(2, 4, 0, 1, 3).reshape(Bdim, T, D)
